# Initial kernel scaffold; baseline (speedup 1.0000x reference)
import jax
import jax.numpy as jnp
from jax.experimental import pallas as pl


def kernel(xs, word_emb, pos_emb, type_emb, ln_gamma, ln_beta, proj_W, proj_b):
    raise NotImplementedError("write your pallas kernel here")


# speedup vs baseline: 2.9328x; 2.9328x over previous
import functools

import jax
import jax.numpy as jnp
from jax import lax
from jax.experimental import pallas as pl
from jax.experimental.pallas import tpu as pltpu
from jax.experimental.pallas import tpu_sc as plsc

_VOCAB = 30522
_EMB = 128
_HID = 256
_TYPE_VOCAB = 2
_B = 128
_S = 512
_NTOK = _B * _S
_LN_EPS = 1e-12

_NC = 2
_NS = 16
_NW = _NC * _NS
_ROWS_PER_W = _NTOK // _NW
_CHUNK = 128
_NCHUNK = _ROWS_PER_W // _CHUNK


def _sc_gather_body(table_hbm, idx_hbm, out_hbm, idx_v, rows_v, sem):
    wid = lax.axis_index("s") * _NC + lax.axis_index("c")
    base = wid * _ROWS_PER_W

    def chunk_body(i, carry):
        off = base + i * _CHUNK
        pltpu.sync_copy(idx_hbm.at[pl.ds(off, _CHUNK)], idx_v)
        pltpu.async_copy(table_hbm.at[idx_v], rows_v, sem).wait()
        pltpu.sync_copy(rows_v, out_hbm.at[pl.ds(off, _CHUNK)])
        return carry

    lax.fori_loop(0, _NCHUNK, chunk_body, 0)


@functools.cache
def _sc_gather():
    return pl.kernel(
        _sc_gather_body,
        out_type=jax.ShapeDtypeStruct((_NTOK, _EMB), jnp.float32),
        mesh=plsc.VectorSubcoreMesh(core_axis_name="c", subcore_axis_name="s"),
        scratch_types=[
            pltpu.VMEM((_CHUNK,), jnp.int32),
            pltpu.VMEM((_CHUNK, _EMB), jnp.float32),
            pltpu.SemaphoreType.DMA,
        ],
    )


_SEQ_BLK = 4


def _tc_body(e_ref, pos_ref, type_ref, gamma_ref, beta_ref, w_ref, b_ref, out_ref):
    t = type_ref[...]
    e = e_ref[...] + pos_ref[...][None] + t[0][None, None, :]
    mu = jnp.mean(e, axis=-1, keepdims=True)
    d = e - mu
    var = jnp.mean(d * d, axis=-1, keepdims=True)
    n = d * lax.rsqrt(var + _LN_EPS)
    n = n * gamma_ref[...][None, None, :] + beta_ref[...][None, None, :]
    out = lax.dot_general(n, w_ref[...], (((2,), (0,)), ((), ())),
                          preferred_element_type=jnp.float32)
    out_ref[...] = out + b_ref[...][None, None, :]


def _tc_fuse(gathered, pos_emb, type_emb, ln_gamma, ln_beta, proj_W, proj_b):
    e3 = gathered.reshape(_B, _S, _EMB)
    return pl.pallas_call(
        _tc_body,
        grid=(_B // _SEQ_BLK,),
        in_specs=[
            pl.BlockSpec((_SEQ_BLK, _S, _EMB), lambda i: (i, 0, 0)),
            pl.BlockSpec((_S, _EMB), lambda i: (0, 0)),
            pl.BlockSpec((_TYPE_VOCAB, _EMB), lambda i: (0, 0)),
            pl.BlockSpec((_EMB,), lambda i: (0,)),
            pl.BlockSpec((_EMB,), lambda i: (0,)),
            pl.BlockSpec((_EMB, _HID), lambda i: (0, 0)),
            pl.BlockSpec((_HID,), lambda i: (0,)),
        ],
        out_specs=pl.BlockSpec((_SEQ_BLK, _S, _HID), lambda i: (i, 0, 0)),
        out_shape=jax.ShapeDtypeStruct((_B, _S, _HID), jnp.float32),
    )(e3, pos_emb, type_emb, ln_gamma, ln_beta, proj_W, proj_b)


def kernel(xs, word_emb, pos_emb, type_emb, ln_gamma, ln_beta, proj_W, proj_b):
    idx = xs.reshape(_NTOK)
    gathered = _sc_gather()(word_emb, idx)
    return _tc_fuse(gathered, pos_emb, type_emb, ln_gamma, ln_beta, proj_W, proj_b)


# speedup vs baseline: 3.4392x; 1.1727x over previous
import functools

import jax
import jax.numpy as jnp
from jax import lax
from jax.experimental import pallas as pl
from jax.experimental.pallas import tpu as pltpu
from jax.experimental.pallas import tpu_sc as plsc

_VOCAB = 30522
_EMB = 128
_HID = 256
_TYPE_VOCAB = 2
_B = 128
_S = 512
_NTOK = _B * _S
_LN_EPS = 1e-12

_NC = 2
_NS = 16
_NW = _NC * _NS
_ROWS_PER_W = _NTOK // _NW
_GATHER = 128
_CH_ROWS = 256
_GPC = _CH_ROWS // _GATHER
_NCHUNK = _ROWS_PER_W // _CH_ROWS


def _sc_gather_body(table_hbm, idx_hbm, out_hbm, idx_v, buf0, buf1,
                    gsem0, gsem1, wsem0, wsem1):
    wid = lax.axis_index("s") * _NC + lax.axis_index("c")
    base = wid * _ROWS_PER_W
    pltpu.sync_copy(idx_hbm.at[pl.ds(base, _ROWS_PER_W)], idx_v)

    bufs = (buf0, buf1)
    gsems = (gsem0, gsem1)
    wsems = (wsem0, wsem1)

    def start_gathers(c):
        b, s = bufs[c % 2], gsems[c % 2]
        return [
            pltpu.async_copy(
                table_hbm.at[idx_v.at[pl.ds(c * _CH_ROWS + g * _GATHER, _GATHER)]],
                b.at[pl.ds(g * _GATHER, _GATHER)], s)
            for g in range(_GPC)
        ]

    gdesc = {0: start_gathers(0)}
    wdesc = {}
    for c in range(_NCHUNK):
        if c + 1 < _NCHUNK:
            if c - 1 >= 0:
                wdesc[c - 1].wait()
            gdesc[c + 1] = start_gathers(c + 1)
        for d in gdesc[c]:
            d.wait()
        wdesc[c] = pltpu.async_copy(
            bufs[c % 2], out_hbm.at[pl.ds(base + c * _CH_ROWS, _CH_ROWS)],
            wsems[c % 2])
    wdesc[_NCHUNK - 2].wait()
    wdesc[_NCHUNK - 1].wait()


@functools.cache
def _sc_gather():
    return pl.kernel(
        _sc_gather_body,
        out_type=jax.ShapeDtypeStruct((_NTOK, _EMB), jnp.float32),
        mesh=plsc.VectorSubcoreMesh(core_axis_name="c", subcore_axis_name="s"),
        scratch_types=[
            pltpu.VMEM((_ROWS_PER_W,), jnp.int32),
            pltpu.VMEM((_CH_ROWS, _EMB), jnp.float32),
            pltpu.VMEM((_CH_ROWS, _EMB), jnp.float32),
            pltpu.SemaphoreType.DMA,
            pltpu.SemaphoreType.DMA,
            pltpu.SemaphoreType.DMA,
            pltpu.SemaphoreType.DMA,
        ],
    )


_SEQ_BLK = 4


def _tc_body(e_ref, pos_ref, type_ref, gamma_ref, beta_ref, w_ref, b_ref, out_ref):
    t = type_ref[...]
    e = e_ref[...] + pos_ref[...][None] + t[0][None, None, :]
    mu = jnp.mean(e, axis=-1, keepdims=True)
    d = e - mu
    var = jnp.mean(d * d, axis=-1, keepdims=True)
    n = d * lax.rsqrt(var + _LN_EPS)
    n = n * gamma_ref[...][None, None, :] + beta_ref[...][None, None, :]
    out = lax.dot_general(n, w_ref[...], (((2,), (0,)), ((), ())),
                          preferred_element_type=jnp.float32)
    out_ref[...] = out + b_ref[...][None, None, :]


def _tc_fuse(gathered, pos_emb, type_emb, ln_gamma, ln_beta, proj_W, proj_b):
    e3 = gathered.reshape(_B, _S, _EMB)
    return pl.pallas_call(
        _tc_body,
        grid=(_B // _SEQ_BLK,),
        in_specs=[
            pl.BlockSpec((_SEQ_BLK, _S, _EMB), lambda i: (i, 0, 0)),
            pl.BlockSpec((_S, _EMB), lambda i: (0, 0)),
            pl.BlockSpec((_TYPE_VOCAB, _EMB), lambda i: (0, 0)),
            pl.BlockSpec((_EMB,), lambda i: (0,)),
            pl.BlockSpec((_EMB,), lambda i: (0,)),
            pl.BlockSpec((_EMB, _HID), lambda i: (0, 0)),
            pl.BlockSpec((_HID,), lambda i: (0,)),
        ],
        out_specs=pl.BlockSpec((_SEQ_BLK, _S, _HID), lambda i: (i, 0, 0)),
        out_shape=jax.ShapeDtypeStruct((_B, _S, _HID), jnp.float32),
    )(e3, pos_emb, type_emb, ln_gamma, ln_beta, proj_W, proj_b)


def kernel(xs, word_emb, pos_emb, type_emb, ln_gamma, ln_beta, proj_W, proj_b):
    idx = xs.reshape(_NTOK)
    gathered = _sc_gather()(word_emb, idx)
    return _tc_fuse(gathered, pos_emb, type_emb, ln_gamma, ln_beta, proj_W, proj_b)


# speedup vs baseline: 3.7946x; 1.1033x over previous
import functools

import jax
import jax.numpy as jnp
from jax import lax
from jax.experimental import pallas as pl
from jax.experimental.pallas import tpu as pltpu
from jax.experimental.pallas import tpu_sc as plsc

_VOCAB = 30522
_EMB = 128
_HID = 256
_TYPE_VOCAB = 2
_B = 128
_S = 512
_NTOK = _B * _S
_LN_EPS = 1e-12

_NC = 2
_NS = 16
_NW = _NC * _NS
_ROWS_PER_W = _NTOK // _NW
_GATHER = 128
_CH_ROWS = 256
_GPC = _CH_ROWS // _GATHER
_NCHUNK = _ROWS_PER_W // _CH_ROWS


def _sc_gather_body(table_hbm, idx_hbm, out_hbm, idx_v, buf0, buf1,
                    gsem0, gsem1, wsem0, wsem1):
    wid = lax.axis_index("s") * _NC + lax.axis_index("c")
    base = wid * _ROWS_PER_W
    pltpu.sync_copy(idx_hbm.at[pl.ds(base, _ROWS_PER_W)], idx_v)

    bufs = (buf0, buf1)
    gsems = (gsem0, gsem1)
    wsems = (wsem0, wsem1)

    def start_gathers(c):
        b, s = bufs[c % 2], gsems[c % 2]
        return [
            pltpu.async_copy(
                table_hbm.at[idx_v.at[pl.ds(c * _CH_ROWS + g * _GATHER, _GATHER)]],
                b.at[pl.ds(g * _GATHER, _GATHER)], s)
            for g in range(_GPC)
        ]

    gdesc = {0: start_gathers(0)}
    wdesc = {}
    for c in range(_NCHUNK):
        if c + 1 < _NCHUNK:
            if c - 1 >= 0:
                wdesc[c - 1].wait()
            gdesc[c + 1] = start_gathers(c + 1)
        for d in gdesc[c]:
            d.wait()
        wdesc[c] = pltpu.async_copy(
            bufs[c % 2], out_hbm.at[pl.ds(base + c * _CH_ROWS, _CH_ROWS)],
            wsems[c % 2])
    wdesc[_NCHUNK - 2].wait()
    wdesc[_NCHUNK - 1].wait()


@functools.cache
def _sc_gather():
    return pl.kernel(
        _sc_gather_body,
        out_type=jax.ShapeDtypeStruct((_NTOK, _EMB), jnp.float32),
        mesh=plsc.VectorSubcoreMesh(core_axis_name="c", subcore_axis_name="s"),
        scratch_types=[
            pltpu.VMEM((_ROWS_PER_W,), jnp.int32),
            pltpu.VMEM((_CH_ROWS, _EMB), jnp.float32),
            pltpu.VMEM((_CH_ROWS, _EMB), jnp.float32),
            pltpu.SemaphoreType.DMA,
            pltpu.SemaphoreType.DMA,
            pltpu.SemaphoreType.DMA,
            pltpu.SemaphoreType.DMA,
        ],
    )


_SEQ_BLK = 8


def _tc_body(e_ref, pos_ref, type_ref, gamma_ref, beta_ref, w_ref, b_ref, out_ref):
    t = type_ref[...]
    e = e_ref[...] + pos_ref[...][None] + t[0][None, None, :]
    mu = jnp.mean(e, axis=-1, keepdims=True)
    d = e - mu
    var = jnp.mean(d * d, axis=-1, keepdims=True)
    n = d * lax.rsqrt(var + _LN_EPS)
    n = n * gamma_ref[...][None, None, :] + beta_ref[...][None, None, :]
    out = lax.dot_general(n, w_ref[...], (((2,), (0,)), ((), ())),
                          preferred_element_type=jnp.float32)
    out_ref[...] = out + b_ref[...][None, None, :]


def _tc_fuse(gathered, pos_emb, type_emb, ln_gamma, ln_beta, proj_W, proj_b):
    e3 = gathered.reshape(_B, _S, _EMB)
    return pl.pallas_call(
        _tc_body,
        grid=(_B // _SEQ_BLK,),
        in_specs=[
            pl.BlockSpec((_SEQ_BLK, _S, _EMB), lambda i: (i, 0, 0)),
            pl.BlockSpec((_S, _EMB), lambda i: (0, 0)),
            pl.BlockSpec((_TYPE_VOCAB, _EMB), lambda i: (0, 0)),
            pl.BlockSpec((_EMB,), lambda i: (0,)),
            pl.BlockSpec((_EMB,), lambda i: (0,)),
            pl.BlockSpec((_EMB, _HID), lambda i: (0, 0)),
            pl.BlockSpec((_HID,), lambda i: (0,)),
        ],
        out_specs=pl.BlockSpec((_SEQ_BLK, _S, _HID), lambda i: (i, 0, 0)),
        out_shape=jax.ShapeDtypeStruct((_B, _S, _HID), jnp.float32),
    )(e3, pos_emb, type_emb, ln_gamma, ln_beta, proj_W, proj_b)


def kernel(xs, word_emb, pos_emb, type_emb, ln_gamma, ln_beta, proj_W, proj_b):
    idx = xs.reshape(_NTOK)
    gathered = _sc_gather()(word_emb, idx)
    return _tc_fuse(gathered, pos_emb, type_emb, ln_gamma, ln_beta, proj_W, proj_b)


# speedup vs baseline: 4.0221x; 1.0600x over previous
import functools

import jax
import jax.numpy as jnp
from jax import lax
from jax.experimental import pallas as pl
from jax.experimental.pallas import tpu as pltpu
from jax.experimental.pallas import tpu_sc as plsc

_VOCAB = 30522
_EMB = 128
_HID = 256
_TYPE_VOCAB = 2
_B = 128
_S = 512
_NTOK = _B * _S
_LN_EPS = 1e-12

_NC = 2
_NS = 16
_NW = _NC * _NS
_ROWS_PER_W = _NTOK // _NW
_GATHER = 128
_CH_ROWS = 256
_GPC = _CH_ROWS // _GATHER
_NCHUNK = _ROWS_PER_W // _CH_ROWS


def _sc_gather_body(table_hbm, idx_hbm, out_hbm, idx_v, buf0, buf1,
                    gsem0, gsem1, wsem0, wsem1):
    wid = lax.axis_index("s") * _NC + lax.axis_index("c")
    base = wid * _ROWS_PER_W
    pltpu.sync_copy(idx_hbm.at[pl.ds(base, _ROWS_PER_W)], idx_v)

    bufs = (buf0, buf1)
    gsems = (gsem0, gsem1)
    wsems = (wsem0, wsem1)

    def start_gathers(c):
        b, s = bufs[c % 2], gsems[c % 2]
        return [
            pltpu.async_copy(
                table_hbm.at[idx_v.at[pl.ds(c * _CH_ROWS + g * _GATHER, _GATHER)]],
                b.at[pl.ds(g * _GATHER, _GATHER)], s)
            for g in range(_GPC)
        ]

    gdesc = {0: start_gathers(0)}
    wdesc = {}
    for c in range(_NCHUNK):
        if c + 1 < _NCHUNK:
            if c - 1 >= 0:
                wdesc[c - 1].wait()
            gdesc[c + 1] = start_gathers(c + 1)
        for d in gdesc[c]:
            d.wait()
        wdesc[c] = pltpu.async_copy(
            bufs[c % 2], out_hbm.at[pl.ds(base + c * _CH_ROWS, _CH_ROWS)],
            wsems[c % 2])
    wdesc[_NCHUNK - 2].wait()
    wdesc[_NCHUNK - 1].wait()


@functools.cache
def _sc_gather():
    return pl.kernel(
        _sc_gather_body,
        out_type=jax.ShapeDtypeStruct((_NTOK, _EMB), jnp.float32),
        mesh=plsc.VectorSubcoreMesh(core_axis_name="c", subcore_axis_name="s"),
        scratch_types=[
            pltpu.VMEM((_ROWS_PER_W,), jnp.int32),
            pltpu.VMEM((_CH_ROWS, _EMB), jnp.float32),
            pltpu.VMEM((_CH_ROWS, _EMB), jnp.float32),
            pltpu.SemaphoreType.DMA,
            pltpu.SemaphoreType.DMA,
            pltpu.SemaphoreType.DMA,
            pltpu.SemaphoreType.DMA,
        ],
    )


_SEQ_BLK = 16


def _tc_body(e_ref, pos_ref, type_ref, gamma_ref, beta_ref, w_ref, b_ref, out_ref):
    t = type_ref[...]
    e = e_ref[...] + pos_ref[...][None] + t[0][None, None, :]
    mu = jnp.mean(e, axis=-1, keepdims=True)
    d = e - mu
    var = jnp.mean(d * d, axis=-1, keepdims=True)
    n = d * lax.rsqrt(var + _LN_EPS)
    n = n * gamma_ref[...][None, None, :] + beta_ref[...][None, None, :]
    out = lax.dot_general(n, w_ref[...], (((2,), (0,)), ((), ())),
                          preferred_element_type=jnp.float32)
    out_ref[...] = out + b_ref[...][None, None, :]


def _tc_fuse(gathered, pos_emb, type_emb, ln_gamma, ln_beta, proj_W, proj_b):
    e3 = gathered.reshape(_B, _S, _EMB)
    return pl.pallas_call(
        _tc_body,
        grid=(_B // _SEQ_BLK,),
        in_specs=[
            pl.BlockSpec((_SEQ_BLK, _S, _EMB), lambda i: (i, 0, 0)),
            pl.BlockSpec((_S, _EMB), lambda i: (0, 0)),
            pl.BlockSpec((_TYPE_VOCAB, _EMB), lambda i: (0, 0)),
            pl.BlockSpec((_EMB,), lambda i: (0,)),
            pl.BlockSpec((_EMB,), lambda i: (0,)),
            pl.BlockSpec((_EMB, _HID), lambda i: (0, 0)),
            pl.BlockSpec((_HID,), lambda i: (0,)),
        ],
        out_specs=pl.BlockSpec((_SEQ_BLK, _S, _HID), lambda i: (i, 0, 0)),
        out_shape=jax.ShapeDtypeStruct((_B, _S, _HID), jnp.float32),
    )(e3, pos_emb, type_emb, ln_gamma, ln_beta, proj_W, proj_b)


def kernel(xs, word_emb, pos_emb, type_emb, ln_gamma, ln_beta, proj_W, proj_b):
    idx = xs.reshape(_NTOK)
    gathered = _sc_gather()(word_emb, idx)
    return _tc_fuse(gathered, pos_emb, type_emb, ln_gamma, ln_beta, proj_W, proj_b)
